# baseline (speedup 1.0000x reference)
import functools

import jax
import jax.numpy as jnp
from jax import lax
from jax.experimental import pallas as pl
from jax.experimental.pallas import tpu as pltpu
from jax.experimental.pallas import tpu_sc as plsc

NENTITY = 1000000
EDIM = 64
NRELATION = 64
BATCH = 16384
PAIRW = 2 * EDIM

VB = 2048
GROUP = 4 * VB
NPACK = (NENTITY + GROUP - 1) // GROUP
PROWS = NPACK * VB
_LASTB = (NENTITY - 1) // VB

NC = 2
NS = 16
NW = NC * NS
ROWS_PER_W = BATCH // NW
CHUNK = 128
NCHUNK = ROWS_PER_W // CHUNK

TC_BLOCK = 2048

_HI = -65536


def _bf16_hi_bits(x):
    r = x.astype(jnp.bfloat16).astype(jnp.float32)
    return lax.bitcast_convert_type(r, jnp.int32) & _HI


def _pack_pair(a, b):
    bu = lax.bitcast_convert_type(_bf16_hi_bits(b), jnp.uint32)
    blo = lax.bitcast_convert_type(jnp.right_shift(bu, 16), jnp.int32)
    bits = _bf16_hi_bits(a) | blo
    return lax.bitcast_convert_type(bits, jnp.float32)


def _pack_body(a_ref, b_ref, c_ref, d_ref, out_ref):
    at = jnp.transpose(a_ref[...], (1, 0))
    bt = jnp.transpose(b_ref[...], (1, 0))
    ct = jnp.transpose(c_ref[...], (1, 0))
    dt = jnp.transpose(d_ref[...], (1, 0))
    w1 = _pack_pair(at, bt)
    w2 = _pack_pair(ct, dt)
    out_ref[...] = jnp.concatenate([w1, w2], axis=1)


def _tc_pack(embT):
    def spec(q):
        return pl.BlockSpec(
            (EDIM, VB), lambda i: (0, jnp.minimum(4 * i + q, _LASTB)))
    return pl.pallas_call(
        _pack_body,
        grid=(NPACK,),
        in_specs=[spec(0), spec(1), spec(2), spec(3)],
        out_specs=pl.BlockSpec((VB, PAIRW), lambda i: (i, 0)),
        out_shape=jax.ShapeDtypeStruct((PROWS, PAIRW), jnp.float32),
    )(embT, embT, embT, embT)


def _sc_gather_body(h_hbm, t_hbm, emb_hbm, gh_hbm, gt_hbm,
                    hidx_v, tidx_v, bufh, buft, semh, semt):
    wid = lax.axis_index("s") * NC + lax.axis_index("c")
    base = wid * ROWS_PER_W
    pltpu.sync_copy(h_hbm.at[pl.ds(wid * NCHUNK, NCHUNK)], hidx_v)
    pltpu.sync_copy(t_hbm.at[pl.ds(wid * NCHUNK, NCHUNK)], tidx_v)
    for c in range(NCHUNK):
        wh = pltpu.async_copy(emb_hbm.at[hidx_v.at[c]], bufh, semh)
        wt_ = pltpu.async_copy(emb_hbm.at[tidx_v.at[c]], buft, semt)
        wh.wait()
        pltpu.sync_copy(bufh, gh_hbm.at[pl.ds(base + c * CHUNK, CHUNK)])
        wt_.wait()
        pltpu.sync_copy(buft, gt_hbm.at[pl.ds(base + c * CHUNK, CHUNK)])


def _sc_gather(h2, t2, packed):
    mesh = plsc.VectorSubcoreMesh(core_axis_name="c", subcore_axis_name="s")
    f = pl.kernel(
        _sc_gather_body,
        out_type=[
            jax.ShapeDtypeStruct((BATCH, PAIRW), jnp.float32),
            jax.ShapeDtypeStruct((BATCH, PAIRW), jnp.float32),
        ],
        mesh=mesh,
        scratch_types=[
            pltpu.VMEM((NCHUNK, CHUNK), jnp.int32),
            pltpu.VMEM((NCHUNK, CHUNK), jnp.int32),
            pltpu.VMEM((CHUNK, PAIRW), jnp.float32),
            pltpu.VMEM((CHUNK, PAIRW), jnp.float32),
            pltpu.SemaphoreType.DMA,
            pltpu.SemaphoreType.DMA,
        ],
        compiler_params=pltpu.CompilerParams(use_tc_tiling_on_sc=True),
    )
    return f(h2, t2, packed)


def _bcast_col(row, width):
    ones = jnp.ones((1, width), dtype=jnp.float32)
    return lax.dot_general(row, ones, (((0,), (0,)), ((), ())),
                           preferred_element_type=jnp.float32)


def _unpack(g_ref, colsel, hilo):
    left = g_ref[:, :EDIM]
    right = g_ref[:, EDIM:]
    word = jnp.where(colsel == 1.0, right, left)
    bits = lax.bitcast_convert_type(word, jnp.int32)
    bits = jnp.where(hilo == 1.0, jnp.left_shift(bits, 16), bits & _HI)
    return lax.bitcast_convert_type(bits, jnp.float32)


def _tc_body(q_ref, er_ref, wt_ref, gh_ref, gt_ref, out_ref):
    qs = q_ref[...]
    hcol = _bcast_col(qs[0:1, :], EDIM)
    hlo = _bcast_col(qs[1:2, :], EDIM)
    tcol = _bcast_col(qs[2:3, :], EDIM)
    tlo = _bcast_col(qs[3:4, :], EDIM)
    relc = _bcast_col(qs[4:5, :], NRELATION)
    eh = _unpack(gh_ref, hcol, hlo)
    et = _unpack(gt_ref, tcol, tlo)
    d = (eh - et).astype(jnp.bfloat16)
    y = jnp.dot(d, wt_ref[...].astype(jnp.bfloat16),
                preferred_element_type=jnp.float32)
    iota = lax.broadcasted_iota(jnp.int32, (1, NRELATION), 1).astype(
        jnp.float32)
    onehot = (relc == iota).astype(jnp.float32)
    e_r = jnp.dot(onehot, er_ref[...], preferred_element_type=jnp.float32)
    z = y + e_r
    out_ref[...] = jnp.sum(z * z, axis=1)


def _tc_score(q5, emb_rel, wt, gh, gt):
    grid = (BATCH // TC_BLOCK,)
    blk = lambda i: (i, 0)
    return pl.pallas_call(
        _tc_body,
        grid=grid,
        in_specs=[
            pl.BlockSpec((5, TC_BLOCK), lambda i: (0, i)),
            pl.BlockSpec((NRELATION, NRELATION), lambda i: (0, 0)),
            pl.BlockSpec((EDIM, EDIM), lambda i: (0, 0)),
            pl.BlockSpec((TC_BLOCK, PAIRW), blk),
            pl.BlockSpec((TC_BLOCK, PAIRW), blk),
        ],
        out_specs=pl.BlockSpec((TC_BLOCK,), lambda i: (i,)),
        out_shape=jax.ShapeDtypeStruct((BATCH,), jnp.float32),
    )(q5, emb_rel, wt, gh, gt)


def _row_of(e):
    return (e // GROUP) * VB + (e % VB)


def kernel(h, rel, t, emb_e, emb_rel, W):
    packed = _tc_pack(emb_e.T)
    h2 = _row_of(h).reshape(BATCH // CHUNK, CHUNK)
    t2 = _row_of(t).reshape(BATCH // CHUNK, CHUNK)
    gh, gt = _sc_gather(h2, t2, packed)
    hq = (h // VB) & 3
    tq = (t // VB) & 3
    q5 = jnp.stack([(hq >> 1).astype(jnp.float32),
                    (hq & 1).astype(jnp.float32),
                    (tq >> 1).astype(jnp.float32),
                    (tq & 1).astype(jnp.float32),
                    rel.astype(jnp.float32)], axis=0)
    return _tc_score(q5, emb_rel, W.T, gh, gt)


# speedup vs baseline: 1.1836x; 1.1836x over previous
import functools

import jax
import jax.numpy as jnp
from jax import lax
from jax.experimental import pallas as pl
from jax.experimental.pallas import tpu as pltpu
from jax.experimental.pallas import tpu_sc as plsc

NENTITY = 1000000
EDIM = 64
NRELATION = 64
BATCH = 16384
PAIRW = 2 * EDIM

VB = 4096
GROUP = 4 * VB
NPACK = (NENTITY + GROUP - 1) // GROUP
PROWS = NPACK * VB
_LASTB = (NENTITY - 1) // VB

NC = 2
NS = 16
NW = NC * NS
ROWS_PER_W = BATCH // NW
CHUNK = 128
NCHUNK = ROWS_PER_W // CHUNK

TC_BLOCK = 2048

_HI = -65536


def _bf16_hi_bits(x):
    r = x.astype(jnp.bfloat16).astype(jnp.float32)
    return lax.bitcast_convert_type(r, jnp.int32) & _HI


def _pack_pair(a, b):
    bu = lax.bitcast_convert_type(_bf16_hi_bits(b), jnp.uint32)
    blo = lax.bitcast_convert_type(jnp.right_shift(bu, 16), jnp.int32)
    bits = _bf16_hi_bits(a) | blo
    return lax.bitcast_convert_type(bits, jnp.float32)


def _pack_body(a_ref, b_ref, c_ref, d_ref, out_ref):
    at = jnp.transpose(a_ref[...], (1, 0))
    bt = jnp.transpose(b_ref[...], (1, 0))
    ct = jnp.transpose(c_ref[...], (1, 0))
    dt = jnp.transpose(d_ref[...], (1, 0))
    w1 = _pack_pair(at, bt)
    w2 = _pack_pair(ct, dt)
    out_ref[...] = jnp.concatenate([w1, w2], axis=1)


def _tc_pack(embT):
    def spec(q):
        return pl.BlockSpec(
            (EDIM, VB), lambda i: (0, jnp.minimum(4 * i + q, _LASTB)))
    return pl.pallas_call(
        _pack_body,
        grid=(NPACK,),
        in_specs=[spec(0), spec(1), spec(2), spec(3)],
        out_specs=pl.BlockSpec((VB, PAIRW), lambda i: (i, 0)),
        out_shape=jax.ShapeDtypeStruct((PROWS, PAIRW), jnp.float32),
    )(embT, embT, embT, embT)


def _sc_gather_body(h_hbm, t_hbm, emb_hbm, gh_hbm, gt_hbm,
                    hidx_v, tidx_v, bufh0, bufh1, buft0, buft1,
                    semh, semt, semo):
    wid = lax.axis_index("s") * NC + lax.axis_index("c")
    base = wid * ROWS_PER_W
    pltpu.sync_copy(h_hbm.at[pl.ds(wid * NCHUNK, NCHUNK)], hidx_v)
    pltpu.sync_copy(t_hbm.at[pl.ds(wid * NCHUNK, NCHUNK)], tidx_v)
    bufh = [bufh0, bufh1]
    buft = [buft0, buft1]
    gw = [pltpu.async_copy(emb_hbm.at[hidx_v.at[0]], bufh[0], semh),
          pltpu.async_copy(emb_hbm.at[tidx_v.at[0]], buft[0], semt)]
    outw = []
    for c in range(NCHUNK):
        b = c & 1
        nw = []
        if c + 1 < NCHUNK:
            nw = [pltpu.async_copy(
                      emb_hbm.at[hidx_v.at[c + 1]], bufh[1 - b], semh),
                  pltpu.async_copy(
                      emb_hbm.at[tidx_v.at[c + 1]], buft[1 - b], semt)]
        for w in gw:
            w.wait()
        gw = nw
        for w in outw:
            w.wait()
        outw = [pltpu.async_copy(
                    bufh[b], gh_hbm.at[pl.ds(base + c * CHUNK, CHUNK)], semo),
                pltpu.async_copy(
                    buft[b], gt_hbm.at[pl.ds(base + c * CHUNK, CHUNK)], semo)]
    for w in outw:
        w.wait()


def _sc_gather(h2, t2, packed):
    mesh = plsc.VectorSubcoreMesh(core_axis_name="c", subcore_axis_name="s")
    f = pl.kernel(
        _sc_gather_body,
        out_type=[
            jax.ShapeDtypeStruct((BATCH, PAIRW), jnp.float32),
            jax.ShapeDtypeStruct((BATCH, PAIRW), jnp.float32),
        ],
        mesh=mesh,
        scratch_types=[
            pltpu.VMEM((NCHUNK, CHUNK), jnp.int32),
            pltpu.VMEM((NCHUNK, CHUNK), jnp.int32),
            pltpu.VMEM((CHUNK, PAIRW), jnp.float32),
            pltpu.VMEM((CHUNK, PAIRW), jnp.float32),
            pltpu.VMEM((CHUNK, PAIRW), jnp.float32),
            pltpu.VMEM((CHUNK, PAIRW), jnp.float32),
            pltpu.SemaphoreType.DMA,
            pltpu.SemaphoreType.DMA,
            pltpu.SemaphoreType.DMA,
        ],
        compiler_params=pltpu.CompilerParams(use_tc_tiling_on_sc=True),
    )
    return f(h2, t2, packed)


def _bcast_col(row, width):
    ones = jnp.ones((1, width), dtype=jnp.float32)
    return lax.dot_general(row, ones, (((0,), (0,)), ((), ())),
                           preferred_element_type=jnp.float32)


def _unpack(g_ref, colsel, hilo):
    left = g_ref[:, :EDIM]
    right = g_ref[:, EDIM:]
    word = jnp.where(colsel == 1.0, right, left)
    bits = lax.bitcast_convert_type(word, jnp.int32)
    bits = jnp.where(hilo == 1.0, jnp.left_shift(bits, 16), bits & _HI)
    return lax.bitcast_convert_type(bits, jnp.float32)


def _tc_body(q_ref, er_ref, wt_ref, gh_ref, gt_ref, out_ref):
    qs = q_ref[...]
    hcol = _bcast_col(qs[0:1, :], EDIM)
    hlo = _bcast_col(qs[1:2, :], EDIM)
    tcol = _bcast_col(qs[2:3, :], EDIM)
    tlo = _bcast_col(qs[3:4, :], EDIM)
    relc = _bcast_col(qs[4:5, :], NRELATION)
    eh = _unpack(gh_ref, hcol, hlo)
    et = _unpack(gt_ref, tcol, tlo)
    d = (eh - et).astype(jnp.bfloat16)
    y = jnp.dot(d, wt_ref[...].astype(jnp.bfloat16),
                preferred_element_type=jnp.float32)
    iota = lax.broadcasted_iota(jnp.int32, (1, NRELATION), 1).astype(
        jnp.float32)
    onehot = (relc == iota).astype(jnp.float32)
    e_r = jnp.dot(onehot, er_ref[...], preferred_element_type=jnp.float32)
    z = y + e_r
    out_ref[...] = jnp.sum(z * z, axis=1)


def _tc_score(q5, emb_rel, wt, gh, gt):
    grid = (BATCH // TC_BLOCK,)
    blk = lambda i: (i, 0)
    return pl.pallas_call(
        _tc_body,
        grid=grid,
        in_specs=[
            pl.BlockSpec((5, TC_BLOCK), lambda i: (0, i)),
            pl.BlockSpec((NRELATION, NRELATION), lambda i: (0, 0)),
            pl.BlockSpec((EDIM, EDIM), lambda i: (0, 0)),
            pl.BlockSpec((TC_BLOCK, PAIRW), blk),
            pl.BlockSpec((TC_BLOCK, PAIRW), blk),
        ],
        out_specs=pl.BlockSpec((TC_BLOCK,), lambda i: (i,)),
        out_shape=jax.ShapeDtypeStruct((BATCH,), jnp.float32),
    )(q5, emb_rel, wt, gh, gt)


def _row_of(e):
    return (e // GROUP) * VB + (e % VB)


def kernel(h, rel, t, emb_e, emb_rel, W):
    packed = _tc_pack(emb_e.T)
    h2 = _row_of(h).reshape(BATCH // CHUNK, CHUNK)
    t2 = _row_of(t).reshape(BATCH // CHUNK, CHUNK)
    gh, gt = _sc_gather(h2, t2, packed)
    hq = (h // VB) & 3
    tq = (t // VB) & 3
    q5 = jnp.stack([(hq >> 1).astype(jnp.float32),
                    (hq & 1).astype(jnp.float32),
                    (tq >> 1).astype(jnp.float32),
                    (tq & 1).astype(jnp.float32),
                    rel.astype(jnp.float32)], axis=0)
    return _tc_score(q5, emb_rel, W.T, gh, gt)
